# baseline (speedup 1.0000x reference)
import jax
import jax.numpy as jnp
from jax import lax
from jax.experimental import pallas as pl
from jax.experimental.pallas import tpu as pltpu
from jax.experimental.pallas import tpu_sc as plsc

HIDDEN = 64
BATCH = 16384
NUM_CLASSES = 1000000
NUM_CORES = 2
NUM_SUBCORES = 16
NW = NUM_CORES * NUM_SUBCORES
B_PER_W = BATCH // NW
CHUNKS = B_PER_W // 16
NBUF = 8


def _sc_body(labels_hbm, tablet_hbm, out_hbm, lab_v, buf_v, out_v, sems):
    wid = lax.axis_index("s") * NUM_CORES + lax.axis_index("c")
    blk = pl.multiple_of((wid // 2) * 1024, 1024)
    pltpu.sync_copy(labels_hbm.at[pl.ds(blk, 1024)], lab_v)
    off = (wid % 2) * 512

    lanes = lax.iota(jnp.int32, 16)

    def fire(l, sl):
        start = pl.multiple_of((l >> 7) * 128, 128)
        pltpu.async_copy(
            tablet_hbm.at[:, pl.ds(start, 128)], buf_v.at[sl], sems.at[sl]
        )

    lv0 = lab_v[pl.ds(off, 16)]
    for j in range(NBUF):
        fire(lv0[j], j)

    def load_cols(l, sl):
        pltpu.make_async_copy(
            tablet_hbm.at[:, pl.ds(0, 128)], buf_v.at[sl], sems.at[sl]
        ).wait()
        colv = jnp.full((16,), l & 127, jnp.int32)
        sv = jnp.full((16,), sl, jnp.int32)
        return [
            plsc.load_gather(buf_v, [sv, lanes + 16 * k, colv])
            for k in range(4)
        ]

    def scale_store(vs, iv):
        n2 = jnp.sum(vs[0] * vs[0] + vs[1] * vs[1] + vs[2] * vs[2]
                     + vs[3] * vs[3])
        acc = jnp.full((16,), n2, jnp.float32)
        xi = plsc.bitcast(acc, jnp.int32)
        y = plsc.bitcast(jnp.int32(0x5F3759DF) - (xi >> 1), jnp.float32)
        for _ in range(3):
            y = y * (1.5 - 0.5 * acc * y * y)
        scale = jnp.minimum(y, 1.0)
        for k in range(4):
            plsc.store_scatter(out_v, [lanes + 16 * k, iv], vs[k] * scale)

    def chunk(ci, carry):
        lv = lab_v[pl.ds(off + ci * 16, 16)]
        for j in range(16):
            sl = j % NBUF
            iv = jnp.full((16,), ci * 16 + j, jnp.int32)
            vs = load_cols(lv[j], sl)
            if j < NBUF:
                fire(lv[j + NBUF], sl)
            else:
                @pl.when(ci < CHUNKS - 1)
                def _():
                    lvn = lab_v[pl.ds(off + ci * 16 + 16, 16)]
                    fire(lvn[j - NBUF], sl)
            scale_store(vs, iv)
        return carry

    lax.fori_loop(0, CHUNKS, chunk, 0)

    base = wid * B_PER_W
    pltpu.sync_copy(out_v, out_hbm.at[:, pl.ds(base, B_PER_W)])


def kernel(labels, embedding_table):
    labels_i = labels.astype(jnp.int32)
    run = pl.kernel(
        _sc_body,
        out_type=jax.ShapeDtypeStruct((HIDDEN, BATCH), jnp.float32),
        mesh=plsc.VectorSubcoreMesh(core_axis_name="c", subcore_axis_name="s"),
        scratch_types=[
            pltpu.VMEM((1024,), jnp.int32),
            pltpu.VMEM((NBUF, HIDDEN, 128), jnp.float32),
            pltpu.VMEM((HIDDEN, B_PER_W), jnp.float32),
            pltpu.SemaphoreType.DMA((NBUF,)),
        ],
        compiler_params=pltpu.CompilerParams(
            use_tc_tiling_on_sc=True, needs_layout_passes=False
        ),
    )
    out_t = run(labels_i, embedding_table.T)
    return out_t.T
